# initial kernel scaffold (unmeasured)
import jax
import jax.numpy as jnp
from jax import lax
from jax.experimental import pallas as pl
from jax.experimental.pallas import tpu as pltpu


def kernel(
    x,
):
    def body(*refs):
        pass

    out_shape = jax.ShapeDtypeStruct(..., jnp.float32)
    return pl.pallas_call(body, out_shape=out_shape)(...)


# baseline (device time: 797845 ns/iter reference)
import jax
import jax.numpy as jnp
from jax import lax
from jax.experimental import pallas as pl
from jax.experimental.pallas import tpu as pltpu

M = 16384
N_OUT = 1024
H = 512


def _exchange(x):

    def body(x_ref, r_ref, s1, r1, s2, r2):
        X = lax.axis_index("x")
        Y = lax.axis_index("y")
        Z = lax.axis_index("z")

        src_cols = (1 - Y) * N_OUT + Z * H
        rdma1 = pltpu.make_async_remote_copy(
            src_ref=x_ref.at[0, :, pl.ds(src_cols, H)],
            dst_ref=r_ref.at[:, pl.ds(Z * H, H)],
            send_sem=s1,
            recv_sem=r1,
            device_id=(X, 1 - Y, Z),
            device_id_type=pl.DeviceIdType.MESH,
        )
        rdma1.start()
        rdma1.wait()

        rdma2 = pltpu.make_async_remote_copy(
            src_ref=r_ref.at[:, pl.ds(Z * H, H)],
            dst_ref=r_ref.at[:, pl.ds(Z * H, H)],
            send_sem=s2,
            recv_sem=r2,
            device_id=(X, Y, 1 - Z),
            device_id_type=pl.DeviceIdType.MESH,
        )
        rdma2.start()
        rdma2.wait()

    return pl.pallas_call(
        body,
        out_shape=jax.ShapeDtypeStruct((M, N_OUT), jnp.float32),
        in_specs=[pl.BlockSpec(memory_space=pltpu.MemorySpace.HBM)],
        out_specs=pl.BlockSpec(memory_space=pltpu.MemorySpace.HBM),
        scratch_shapes=[
            pltpu.SemaphoreType.DMA,
            pltpu.SemaphoreType.DMA,
            pltpu.SemaphoreType.DMA,
            pltpu.SemaphoreType.DMA,
        ],
    )(x)


def kernel(x):
    r = _exchange(x)
    Y = lax.axis_index("y")
    local = lax.dynamic_slice_in_dim(x[0], Y * N_OUT, N_OUT, axis=1)
    return local + r


# device time: 530587 ns/iter; 1.5037x vs baseline; 1.5037x over previous
import jax
import jax.numpy as jnp
from jax import lax
from jax.experimental import pallas as pl
from jax.experimental.pallas import tpu as pltpu

M = 16384
N_OUT = 1024
H = 512
NC = 16
RC = M // NC


def kernel(x):
    def body(x_ref, out_ref, a_ref, b_ref, abuf, bbuf, lbuf, obuf,
             s1, r1, s2, r2, asem, bsem, lsem, osem):
        X = lax.axis_index("x")
        Y = lax.axis_index("y")
        Z = lax.axis_index("z")
        src_col = (1 - Y) * N_OUT + Z * H
        lcl_col = Y * N_OUT

        def rdma1(c):
            return pltpu.make_async_remote_copy(
                src_ref=x_ref.at[0, pl.ds(c * RC, RC), pl.ds(src_col, H)],
                dst_ref=a_ref.at[c],
                send_sem=s1.at[c],
                recv_sem=r1.at[c],
                device_id=(X, 1 - Y, Z),
                device_id_type=pl.DeviceIdType.MESH,
            )

        def rdma2(c):
            return pltpu.make_async_remote_copy(
                src_ref=a_ref.at[c],
                dst_ref=b_ref.at[c],
                send_sem=s2.at[c],
                recv_sem=r2.at[c],
                device_id=(X, Y, 1 - Z),
                device_id_type=pl.DeviceIdType.MESH,
            )

        def copy_a(c):
            return pltpu.make_async_copy(a_ref.at[c], abuf.at[c % 2], asem.at[c % 2])

        def copy_b(c):
            return pltpu.make_async_copy(b_ref.at[c], bbuf.at[c % 2], bsem.at[c % 2])

        def load_local(c):
            return pltpu.make_async_copy(
                x_ref.at[0, pl.ds(c * RC, RC), pl.ds(lcl_col, N_OUT)],
                lbuf.at[c % 2],
                lsem.at[c % 2],
            )

        def store_out(c):
            return pltpu.make_async_copy(
                obuf.at[c % 2],
                out_ref.at[pl.ds(c * RC, RC), :],
                osem.at[c % 2],
            )

        for c in range(NC):
            rdma1(c).start()
        load_local(0).start()

        for c in range(NC):
            if c + 1 < NC:
                load_local(c + 1).start()
            rdma1(c).wait()
            rdma2(c).start()
            copy_a(c).start()
            rdma2(c).wait()
            copy_b(c).start()
            load_local(c).wait()
            copy_a(c).wait()
            copy_b(c).wait()
            if c >= 2:
                store_out(c - 2).wait()
            l = lbuf[c % 2]
            a = abuf[c % 2]
            b = bbuf[c % 2]

            @pl.when(Z == 0)
            def _():
                obuf[c % 2, :, :H] = l[:, :H] + a
                obuf[c % 2, :, H:] = l[:, H:] + b

            @pl.when(Z == 1)
            def _():
                obuf[c % 2, :, :H] = l[:, :H] + b
                obuf[c % 2, :, H:] = l[:, H:] + a

            store_out(c).start()

        store_out(NC - 2).wait()
        store_out(NC - 1).wait()

    out, _a, _b = pl.pallas_call(
        body,
        out_shape=(
            jax.ShapeDtypeStruct((M, N_OUT), jnp.float32),
            jax.ShapeDtypeStruct((NC, RC, H), jnp.float32),
            jax.ShapeDtypeStruct((NC, RC, H), jnp.float32),
        ),
        in_specs=[pl.BlockSpec(memory_space=pltpu.MemorySpace.HBM)],
        out_specs=(
            pl.BlockSpec(memory_space=pltpu.MemorySpace.HBM),
            pl.BlockSpec(memory_space=pltpu.MemorySpace.HBM),
            pl.BlockSpec(memory_space=pltpu.MemorySpace.HBM),
        ),
        scratch_shapes=[
            pltpu.VMEM((2, RC, H), jnp.float32),
            pltpu.VMEM((2, RC, H), jnp.float32),
            pltpu.VMEM((2, RC, N_OUT), jnp.float32),
            pltpu.VMEM((2, RC, N_OUT), jnp.float32),
            pltpu.SemaphoreType.DMA((NC,)),
            pltpu.SemaphoreType.DMA((NC,)),
            pltpu.SemaphoreType.DMA((NC,)),
            pltpu.SemaphoreType.DMA((NC,)),
            pltpu.SemaphoreType.DMA((2,)),
            pltpu.SemaphoreType.DMA((2,)),
            pltpu.SemaphoreType.DMA((2,)),
            pltpu.SemaphoreType.DMA((2,)),
        ],
    )(x)
    return out


# device time: 444696 ns/iter; 1.7941x vs baseline; 1.1931x over previous
import jax
import jax.numpy as jnp
from jax import lax
from jax.experimental import pallas as pl
from jax.experimental.pallas import tpu as pltpu

M = 16384
N_OUT = 1024
H = 512
NC = 16
RC = M // NC


def kernel(x):
    def body(x_ref, out_ref, a_ref, b_ref, abuf, bbuf, lbuf, obuf,
             s1, r1, s2, r2, asem, bsem, lsem, osem):
        X = lax.axis_index("x")
        Y = lax.axis_index("y")
        Z = lax.axis_index("z")
        src_col = (1 - Y) * N_OUT + Z * H
        lcl_col = Y * N_OUT

        def rdma1(c):
            return pltpu.make_async_remote_copy(
                src_ref=x_ref.at[0, pl.ds(c * RC, RC), pl.ds(src_col, H)],
                dst_ref=a_ref.at[c],
                send_sem=s1.at[c],
                recv_sem=r1.at[c],
                device_id=(X, 1 - Y, Z),
                device_id_type=pl.DeviceIdType.MESH,
            )

        def rdma2(c):
            return pltpu.make_async_remote_copy(
                src_ref=a_ref.at[c],
                dst_ref=b_ref.at[c],
                send_sem=s2.at[c],
                recv_sem=r2.at[c],
                device_id=(X, Y, 1 - Z),
                device_id_type=pl.DeviceIdType.MESH,
            )

        def copy_a(c):
            return pltpu.make_async_copy(a_ref.at[c], abuf.at[c % 4], asem.at[c % 4])

        def copy_b(c):
            return pltpu.make_async_copy(b_ref.at[c], bbuf.at[c % 4], bsem.at[c % 4])

        def load_local(c):
            return pltpu.make_async_copy(
                x_ref.at[0, pl.ds(c * RC, RC), pl.ds(lcl_col, N_OUT)],
                lbuf.at[c % 4],
                lsem.at[c % 4],
            )

        def store_out(c):
            return pltpu.make_async_copy(
                obuf.at[c % 2],
                out_ref.at[pl.ds(c * RC, RC), :],
                osem.at[c % 2],
            )

        for c in range(NC):
            rdma1(c).start()

        for c in range(NC + 2):
            if c < NC:
                load_local(c).start()
                rdma1(c).wait_recv()
                rdma2(c).start()
                copy_a(c).start()
                rdma1(c).wait_send()
            cb = c - 1
            if 0 <= cb < NC:
                rdma2(cb).wait_recv()
                copy_b(cb).start()
                rdma2(cb).wait_send()
            cc = c - 2
            if cc >= 0:
                load_local(cc).wait()
                copy_a(cc).wait()
                copy_b(cc).wait()
                if cc >= 2:
                    store_out(cc - 2).wait()
                l = lbuf[cc % 4]
                a = abuf[cc % 4]
                b = bbuf[cc % 4]

                @pl.when(Z == 0)
                def _():
                    obuf[cc % 2, :, :H] = l[:, :H] + a
                    obuf[cc % 2, :, H:] = l[:, H:] + b

                @pl.when(Z == 1)
                def _():
                    obuf[cc % 2, :, :H] = l[:, :H] + b
                    obuf[cc % 2, :, H:] = l[:, H:] + a

                store_out(cc).start()

        store_out(NC - 2).wait()
        store_out(NC - 1).wait()

    out, _a, _b = pl.pallas_call(
        body,
        out_shape=(
            jax.ShapeDtypeStruct((M, N_OUT), jnp.float32),
            jax.ShapeDtypeStruct((NC, RC, H), jnp.float32),
            jax.ShapeDtypeStruct((NC, RC, H), jnp.float32),
        ),
        in_specs=[pl.BlockSpec(memory_space=pltpu.MemorySpace.HBM)],
        out_specs=(
            pl.BlockSpec(memory_space=pltpu.MemorySpace.HBM),
            pl.BlockSpec(memory_space=pltpu.MemorySpace.HBM),
            pl.BlockSpec(memory_space=pltpu.MemorySpace.HBM),
        ),
        scratch_shapes=[
            pltpu.VMEM((4, RC, H), jnp.float32),
            pltpu.VMEM((4, RC, H), jnp.float32),
            pltpu.VMEM((4, RC, N_OUT), jnp.float32),
            pltpu.VMEM((2, RC, N_OUT), jnp.float32),
            pltpu.SemaphoreType.DMA((NC,)),
            pltpu.SemaphoreType.DMA((NC,)),
            pltpu.SemaphoreType.DMA((NC,)),
            pltpu.SemaphoreType.DMA((NC,)),
            pltpu.SemaphoreType.DMA((4,)),
            pltpu.SemaphoreType.DMA((4,)),
            pltpu.SemaphoreType.DMA((4,)),
            pltpu.SemaphoreType.DMA((2,)),
        ],
        compiler_params=pltpu.CompilerParams(
            vmem_limit_bytes=56 * 1024 * 1024,
        ),
    )(x)
    return out


# device time: 426891 ns/iter; 1.8690x vs baseline; 1.0417x over previous
import jax
import jax.numpy as jnp
from jax import lax
from jax.experimental import pallas as pl
from jax.experimental.pallas import tpu as pltpu

M = 16384
N_OUT = 1024
H = 512
NC = 32
RC = M // NC


def kernel(x):
    def body(x_ref, out_ref, a_ref, b_ref, abuf, bbuf, lbuf, obuf,
             s1, r1, s2, r2, asem, bsem, lsem, osem):
        X = lax.axis_index("x")
        Y = lax.axis_index("y")
        Z = lax.axis_index("z")
        src_col = (1 - Y) * N_OUT + Z * H
        lcl_col = Y * N_OUT

        def rdma1(c):
            return pltpu.make_async_remote_copy(
                src_ref=x_ref.at[0, pl.ds(c * RC, RC), pl.ds(src_col, H)],
                dst_ref=a_ref.at[c],
                send_sem=s1.at[c],
                recv_sem=r1.at[c],
                device_id=(X, 1 - Y, Z),
                device_id_type=pl.DeviceIdType.MESH,
            )

        def rdma2(c):
            return pltpu.make_async_remote_copy(
                src_ref=a_ref.at[c],
                dst_ref=b_ref.at[c],
                send_sem=s2.at[c],
                recv_sem=r2.at[c],
                device_id=(X, Y, 1 - Z),
                device_id_type=pl.DeviceIdType.MESH,
            )

        def copy_a(c):
            return pltpu.make_async_copy(a_ref.at[c], abuf.at[c % 4], asem.at[c % 4])

        def copy_b(c):
            return pltpu.make_async_copy(b_ref.at[c], bbuf.at[c % 4], bsem.at[c % 4])

        def load_local(c):
            return pltpu.make_async_copy(
                x_ref.at[0, pl.ds(c * RC, RC), pl.ds(lcl_col, N_OUT)],
                lbuf.at[c % 4],
                lsem.at[c % 4],
            )

        def store_out(c):
            return pltpu.make_async_copy(
                obuf.at[c % 2],
                out_ref.at[pl.ds(c * RC, RC), :],
                osem.at[c % 2],
            )

        barrier_sem = pltpu.get_barrier_semaphore()
        for nbr in [(X, 1 - Y, Z), (X, Y, 1 - Z)]:
            pl.semaphore_signal(
                barrier_sem, inc=1,
                device_id=nbr, device_id_type=pl.DeviceIdType.MESH,
            )
        pl.semaphore_wait(barrier_sem, 2)

        for c in range(NC):
            rdma1(c).start()

        for c in range(NC + 2):
            if c < NC:
                load_local(c).start()
                rdma1(c).wait_recv()
                rdma2(c).start()
                copy_a(c).start()
                rdma1(c).wait_send()
            cb = c - 1
            if 0 <= cb < NC:
                rdma2(cb).wait_recv()
                copy_b(cb).start()
                rdma2(cb).wait_send()
            cc = c - 2
            if cc >= 0:
                load_local(cc).wait()
                copy_a(cc).wait()
                copy_b(cc).wait()
                if cc >= 2:
                    store_out(cc - 2).wait()
                l = lbuf[cc % 4]
                a = abuf[cc % 4]
                b = bbuf[cc % 4]

                @pl.when(Z == 0)
                def _():
                    obuf[cc % 2, :, :H] = l[:, :H] + a
                    obuf[cc % 2, :, H:] = l[:, H:] + b

                @pl.when(Z == 1)
                def _():
                    obuf[cc % 2, :, :H] = l[:, :H] + b
                    obuf[cc % 2, :, H:] = l[:, H:] + a

                store_out(cc).start()

        store_out(NC - 2).wait()
        store_out(NC - 1).wait()

    out, _a, _b = pl.pallas_call(
        body,
        out_shape=(
            jax.ShapeDtypeStruct((M, N_OUT), jnp.float32),
            jax.ShapeDtypeStruct((NC, RC, H), jnp.float32),
            jax.ShapeDtypeStruct((NC, RC, H), jnp.float32),
        ),
        in_specs=[pl.BlockSpec(memory_space=pltpu.MemorySpace.HBM)],
        out_specs=(
            pl.BlockSpec(memory_space=pltpu.MemorySpace.HBM),
            pl.BlockSpec(memory_space=pltpu.MemorySpace.HBM),
            pl.BlockSpec(memory_space=pltpu.MemorySpace.HBM),
        ),
        scratch_shapes=[
            pltpu.VMEM((4, RC, H), jnp.float32),
            pltpu.VMEM((4, RC, H), jnp.float32),
            pltpu.VMEM((4, RC, N_OUT), jnp.float32),
            pltpu.VMEM((2, RC, N_OUT), jnp.float32),
            pltpu.SemaphoreType.DMA((NC,)),
            pltpu.SemaphoreType.DMA((NC,)),
            pltpu.SemaphoreType.DMA((NC,)),
            pltpu.SemaphoreType.DMA((NC,)),
            pltpu.SemaphoreType.DMA((4,)),
            pltpu.SemaphoreType.DMA((4,)),
            pltpu.SemaphoreType.DMA((4,)),
            pltpu.SemaphoreType.DMA((2,)),
        ],
        compiler_params=pltpu.CompilerParams(
            vmem_limit_bytes=56 * 1024 * 1024,
            collective_id=0,
        ),
    )(x)
    return out


# device time: 358395 ns/iter; 2.2262x vs baseline; 1.1911x over previous
import jax
import jax.numpy as jnp
from jax import lax
from jax.experimental import pallas as pl
from jax.experimental.pallas import tpu as pltpu

M = 16384
N_OUT = 1024
QR = M // 4
RQ = 512
QN = QR // RQ
HN = QN // 2


def kernel(x):
    def body(x_ref, out_ref, r_ref, qbuf, lbuf, obuf,
             sy, ry, sx, rx, sz, rz, srz, rdz, srx, rdx,
             qsem, lsem, osem):
        X = lax.axis_index("x")
        Y = lax.axis_index("y")
        Z = lax.axis_index("z")
        lcl_col = Y * N_OUT
        rem_col = (1 - Y) * N_OUT
        j_me = 2 * X + Z
        j_x = 2 * (1 - X) + Z
        j_z = 2 * X + (1 - Z)
        j_d = 2 * (1 - X) + (1 - Z)
        p_me = j_me * QR
        p_x = j_x * QR
        p_z = j_z * QR
        p_d = j_d * QR

        def rdma_y(k):
            return pltpu.make_async_remote_copy(
                src_ref=x_ref.at[0, pl.ds(p_me + k * RQ, RQ), pl.ds(rem_col, N_OUT)],
                dst_ref=r_ref.at[pl.ds(p_me + k * RQ, RQ), :],
                send_sem=sy.at[k],
                recv_sem=ry.at[k],
                device_id=(X, 1 - Y, Z),
                device_id_type=pl.DeviceIdType.MESH,
            )

        def rdma_x(k):
            return pltpu.make_async_remote_copy(
                src_ref=r_ref.at[pl.ds(p_me + k * RQ, RQ), :],
                dst_ref=r_ref.at[pl.ds(p_me + k * RQ, RQ), :],
                send_sem=sx.at[k],
                recv_sem=rx.at[k],
                device_id=(1 - X, Y, Z),
                device_id_type=pl.DeviceIdType.MESH,
            )

        def rdma_z(k):
            return pltpu.make_async_remote_copy(
                src_ref=r_ref.at[pl.ds(p_me + k * RQ, RQ), :],
                dst_ref=r_ref.at[pl.ds(p_me + k * RQ, RQ), :],
                send_sem=sz.at[k],
                recv_sem=rz.at[k],
                device_id=(X, Y, 1 - Z),
                device_id_type=pl.DeviceIdType.MESH,
            )

        def rdma_rz(k):
            return pltpu.make_async_remote_copy(
                src_ref=r_ref.at[pl.ds(p_x + k * RQ, RQ), pl.ds(0, N_OUT // 2)],
                dst_ref=r_ref.at[pl.ds(p_x + k * RQ, RQ), pl.ds(0, N_OUT // 2)],
                send_sem=srz.at[k],
                recv_sem=rdz.at[k],
                device_id=(X, Y, 1 - Z),
                device_id_type=pl.DeviceIdType.MESH,
            )

        def rdma_rx(k):
            return pltpu.make_async_remote_copy(
                src_ref=r_ref.at[
                    pl.ds(p_z + k * RQ, RQ), pl.ds(N_OUT // 2, N_OUT // 2)
                ],
                dst_ref=r_ref.at[
                    pl.ds(p_z + k * RQ, RQ), pl.ds(N_OUT // 2, N_OUT // 2)
                ],
                send_sem=srx.at[k],
                recv_sem=rdx.at[k],
                device_id=(1 - X, Y, Z),
                device_id_type=pl.DeviceIdType.MESH,
            )

        barrier_sem = pltpu.get_barrier_semaphore()
        for nbr in [(X, 1 - Y, Z), (1 - X, Y, Z), (X, Y, 1 - Z)]:
            pl.semaphore_signal(
                barrier_sem, inc=1,
                device_id=nbr, device_id_type=pl.DeviceIdType.MESH,
            )
        pl.semaphore_wait(barrier_sem, 3)

        for k in range(QN):
            rdma_y(k).start()

        item_i = [0]
        pending = [[]]

        def start_item(row0, i):
            pltpu.make_async_copy(
                r_ref.at[pl.ds(row0, RQ), :], qbuf.at[i % 8], qsem.at[i % 8]
            ).start()
            pltpu.make_async_copy(
                x_ref.at[0, pl.ds(row0, RQ), pl.ds(lcl_col, N_OUT)],
                lbuf.at[i % 8],
                lsem.at[i % 8],
            ).start()

        def finish_item(row0, i):
            pltpu.make_async_copy(
                r_ref.at[pl.ds(row0, RQ), :], qbuf.at[i % 8], qsem.at[i % 8]
            ).wait()
            pltpu.make_async_copy(
                x_ref.at[0, pl.ds(row0, RQ), pl.ds(lcl_col, N_OUT)],
                lbuf.at[i % 8],
                lsem.at[i % 8],
            ).wait()
            if i >= 4:
                pltpu.make_async_copy(
                    obuf.at[i % 4], out_ref.at[pl.ds(0, RQ), :], osem.at[i % 4]
                ).wait()
            obuf[i % 4] = lbuf[i % 8] + qbuf[i % 8]
            pltpu.make_async_copy(
                obuf.at[i % 4], out_ref.at[pl.ds(row0, RQ), :], osem.at[i % 4]
            ).start()

        for s in range(QN + 3):
            if s < QN:
                rdma_y(s).wait_recv()
                rdma_x(s).start()
                rdma_z(s).start()
                rdma_y(s).wait_send()
            k1 = s - 1
            if 0 <= k1 < QN:
                rdma_x(k1).wait_recv()
                rdma_rz(k1).start()
                rdma_z(k1).wait_recv()
                rdma_rx(k1).start()
                rdma_x(k1).wait_send()
                rdma_z(k1).wait_send()
            k2 = s - 2
            if 0 <= k2 < QN:
                rdma_rz(k2).wait_recv()
                rdma_rx(k2).wait_recv()
                rdma_rz(k2).wait_send()
                rdma_rx(k2).wait_send()

            rows = []
            if 0 <= s - 1 < QN:
                rows.append(p_me + (s - 1) * RQ)
            if 0 <= s - 2 < QN:
                rows.append(p_x + (s - 2) * RQ)
                rows.append(p_z + (s - 2) * RQ)
            if 0 <= s - 3 < QN:
                rows.append(p_d + (s - 3) * RQ)
            new_items = [(row0, item_i[0] + n) for n, row0 in enumerate(rows)]
            item_i[0] += len(rows)
            for row0, i in new_items:
                start_item(row0, i)
            for row0, i in pending[0]:
                finish_item(row0, i)
            pending[0] = new_items

        for row0, i in pending[0]:
            finish_item(row0, i)

        for i in range(max(0, item_i[0] - 4), item_i[0]):
            pltpu.make_async_copy(
                obuf.at[i % 4], out_ref.at[pl.ds(0, RQ), :], osem.at[i % 4]
            ).wait()

    out, _r = pl.pallas_call(
        body,
        out_shape=(
            jax.ShapeDtypeStruct((M, N_OUT), jnp.float32),
            jax.ShapeDtypeStruct((M, N_OUT), jnp.float32),
        ),
        in_specs=[pl.BlockSpec(memory_space=pltpu.MemorySpace.HBM)],
        out_specs=(
            pl.BlockSpec(memory_space=pltpu.MemorySpace.HBM),
            pl.BlockSpec(memory_space=pltpu.MemorySpace.HBM),
        ),
        scratch_shapes=[
            pltpu.VMEM((8, RQ, N_OUT), jnp.float32),
            pltpu.VMEM((8, RQ, N_OUT), jnp.float32),
            pltpu.VMEM((4, RQ, N_OUT), jnp.float32),
            pltpu.SemaphoreType.DMA((QN,)),
            pltpu.SemaphoreType.DMA((QN,)),
            pltpu.SemaphoreType.DMA((QN,)),
            pltpu.SemaphoreType.DMA((QN,)),
            pltpu.SemaphoreType.DMA((QN,)),
            pltpu.SemaphoreType.DMA((QN,)),
            pltpu.SemaphoreType.DMA((QN,)),
            pltpu.SemaphoreType.DMA((QN,)),
            pltpu.SemaphoreType.DMA((QN,)),
            pltpu.SemaphoreType.DMA((QN,)),
            pltpu.SemaphoreType.DMA((8,)),
            pltpu.SemaphoreType.DMA((8,)),
            pltpu.SemaphoreType.DMA((4,)),
        ],
        compiler_params=pltpu.CompilerParams(
            vmem_limit_bytes=56 * 1024 * 1024,
            collective_id=0,
        ),
    )(x)
    return out
